# Initial kernel scaffold; baseline (speedup 1.0000x reference)
import jax
import jax.numpy as jnp
from jax import lax
from jax.experimental import pallas as pl
from jax.experimental.pallas import tpu as pltpu
from jax.experimental.pallas import tpu_sc as plsc

BATCH = 4096
CTX = 200
VOCAB = 100000
DIM = 64
N = BATCH * CTX
NC = 2
NS = 16
NW = NC * NS
PER_W = N // NW
CHUNK = 128
N_CHUNKS = PER_W // CHUNK


def _emb_body(x_hbm, p_hbm, emb_hbm, pos_hbm, out_hbm, xidx, pidx, ebuf, pbuf, sem):
    wid = lax.axis_index("s") * NC + lax.axis_index("c")

    pltpu.sync_copy(x_hbm.at[wid], xidx)
    pltpu.sync_copy(p_hbm.at[wid], pidx)

    def chunk_body(c, _):
        ge = pltpu.async_copy(emb_hbm.at[xidx.at[c]], ebuf, sem)
        gp = pltpu.async_copy(pos_hbm.at[pidx.at[c]], pbuf, sem)
        ge.wait()
        gp.wait()

        def row_body(i, _):
            for j in range(DIM // 16):
                v = pbuf[i, pl.ds(j * 16, 16)]
                plsc.addupdate(ebuf.at[i, pl.ds(j * 16, 16)], v)
            return 0

        lax.fori_loop(0, CHUNK, row_body, 0)
        pltpu.sync_copy(ebuf, out_hbm.at[wid, c])
        return 0

    lax.fori_loop(0, N_CHUNKS, chunk_body, 0)


@jax.jit
def kernel(x, _pos, emb_table, pos_table):
    xf = x.reshape(NW, N_CHUNKS, CHUNK).astype(jnp.int32)
    pf = _pos.reshape(NW, N_CHUNKS, CHUNK).astype(jnp.int32)
    k = pl.kernel(
        _emb_body,
        out_type=jax.ShapeDtypeStruct((NW, N_CHUNKS, CHUNK, DIM), jnp.float32),
        mesh=plsc.VectorSubcoreMesh(core_axis_name="c", subcore_axis_name="s"),
        scratch_types=[
            pltpu.VMEM((N_CHUNKS, CHUNK), jnp.int32),
            pltpu.VMEM((N_CHUNKS, CHUNK), jnp.int32),
            pltpu.VMEM((CHUNK, DIM), jnp.float32),
            pltpu.VMEM((CHUNK, DIM), jnp.float32),
            pltpu.SemaphoreType.DMA,
        ],
    )
    out = k(xf, pf, emb_table, pos_table)
    return out.reshape(BATCH, CTX, DIM)


# speedup vs baseline: 4.5853x; 4.5853x over previous
import jax
import jax.numpy as jnp
from jax import lax
from jax.experimental import pallas as pl
from jax.experimental.pallas import tpu as pltpu
from jax.experimental.pallas import tpu_sc as plsc

BATCH = 4096
CTX = 200
VOCAB = 100000
DIM = 64
N = BATCH * CTX
NC = 2
NS = 16
NW = NC * NS
PER_W = N // NW
CHUNK = 128
N_CHUNKS = PER_W // CHUNK


def _emb_body(x_hbm, p_hbm, emb_hbm, pos_hbm, out_hbm, xidx, pidx, ebuf, pbuf, sem):
    wid = lax.axis_index("s") * NC + lax.axis_index("c")

    pltpu.sync_copy(x_hbm.at[wid], xidx)
    pltpu.sync_copy(p_hbm.at[wid], pidx)

    def chunk_body(c, _):
        ge = pltpu.async_copy(emb_hbm.at[xidx.at[c]], ebuf, sem)
        gp = pltpu.async_copy(pos_hbm.at[pidx.at[c]], pbuf, sem)
        ge.wait()
        gp.wait()

        def row_body(i, _):
            for j in range(DIM // 16):
                v = pbuf[i, pl.ds(j * 16, 16)]
                plsc.addupdate(ebuf.at[i, pl.ds(j * 16, 16)], v)
            return 0

        lax.fori_loop(0, CHUNK, row_body, 0)
        pltpu.sync_copy(ebuf, out_hbm.at[wid, c])
        return 0

    lax.fori_loop(0, N_CHUNKS, chunk_body, 0)


@jax.jit
def kernel(x, _pos, emb_table, pos_table):
    xf = x.reshape(NW, N_CHUNKS, CHUNK).astype(jnp.int32)
    pf = _pos.reshape(NW, N_CHUNKS, CHUNK).astype(jnp.int32)
    k = pl.kernel(
        _emb_body,
        out_type=jax.ShapeDtypeStruct((NW, N_CHUNKS, CHUNK, DIM), jnp.float32),
        mesh=plsc.VectorSubcoreMesh(core_axis_name="c", subcore_axis_name="s"),
        compiler_params=pltpu.CompilerParams(use_tc_tiling_on_sc=False),
        scratch_types=[
            pltpu.VMEM((N_CHUNKS, CHUNK), jnp.int32),
            pltpu.VMEM((N_CHUNKS, CHUNK), jnp.int32),
            pltpu.VMEM((CHUNK, DIM), jnp.float32),
            pltpu.VMEM((CHUNK, DIM), jnp.float32),
            pltpu.SemaphoreType.DMA,
        ],
    )
    out = k(xf, pf, emb_table, pos_table)
    return out.reshape(BATCH, CTX, DIM)
